# baseline (speedup 1.0000x reference)
import functools

import jax
import jax.numpy as jnp
from jax import lax
from jax.experimental import pallas as pl
from jax.experimental.pallas import tpu as pltpu
from jax.experimental.pallas import tpu_sc as plsc


def _dense_body(n, nb, x_ref, W0_ref, Wp_ref, bp_ref, Wu_ref, bu_ref,
                Wr_ref, br_ref, Wh_ref, bh_ref, Wo_ref, W1_ref, v_ref,
                acc_ref, wv_ref):
    p = pl.program_id(0)
    i = pl.program_id(1)
    x = x_ref[...]
    din = x.shape[1]

    @pl.when((p == 0) & (i == 0))
    def _():
        acc_ref[...] = jnp.zeros_like(acc_ref)

    @pl.when(p == 0)
    def _():
        acc_ref[...] += jnp.sum(x, axis=0, keepdims=True)

    @pl.when((p == 0) & (i == nb - 1))
    def _():
        W0 = W0_ref[...]
        mean = acc_ref[...] * (1.0 / n)
        ctx = mean @ Wp_ref[...] + bp_ref[...]
        Wu = Wu_ref[...]
        Wr = Wr_ref[...]
        Wh = Wh_ref[...]
        z = jax.nn.sigmoid(ctx @ Wu[:din] + W0 @ Wu[din:] + bu_ref[...])
        r = jax.nn.sigmoid(ctx @ Wr[:din] + W0 @ Wr[din:] + br_ref[...])
        h_tilde = jnp.tanh(
            ctx @ Wh[:din] + (r * W0) @ Wh[din:] + bh_ref[...])
        W1 = z * W0 + (1.0 - z) * h_tilde
        W1_ref[...] = W1
        wv_ref[...] = W1 @ Wo_ref[...]

    @pl.when(p == 1)
    def _():
        v_ref[...] = x @ wv_ref[...]


def _dense(x, W0, Wp, bp, Wu, bu, Wr, br, Wh, bh, Wo):
    n = x.shape[0]
    din, dh = W0.shape
    nb = 10 if n % 10 == 0 else 1
    blk = n // nb
    full = lambda shape: pl.BlockSpec(shape, lambda p, i: (0, 0))
    return pl.pallas_call(
        functools.partial(_dense_body, n, nb),
        grid=(2, nb),
        in_specs=[
            pl.BlockSpec((blk, din), lambda p, i: (i, 0)),
            full((din, dh)), full((din, din)), full((1, din)),
            full((din + dh, dh)), full((1, dh)),
            full((din + dh, dh)), full((1, dh)),
            full((din + dh, dh)), full((1, dh)),
            full((dh, 1)),
        ],
        out_specs=[
            pl.BlockSpec((din, dh), lambda p, i: (0, 0)),
            pl.BlockSpec((blk, 1), lambda p, i: (i, 0)),
        ],
        out_shape=[
            jax.ShapeDtypeStruct((din, dh), jnp.float32),
            jax.ShapeDtypeStruct((n, 1), jnp.float32),
        ],
        scratch_shapes=[
            pltpu.VMEM((1, din), jnp.float32),
            pltpu.VMEM((din, 1), jnp.float32),
        ],
    )(x, W0, Wp, bp.reshape(1, -1), Wu, bu.reshape(1, -1),
      Wr, br.reshape(1, -1), Wh, bh.reshape(1, -1), Wo)


_NC = 2
_NS = 16
_NW = _NC * _NS
_LANE = 16
_CB = 128
_K = 8


def _make_sc(n_out_pad, ch0, ch1):
    zl = n_out_pad // _NS
    sl = n_out_pad // _NW
    ch = ch0
    mesh = plsc.VectorSubcoreMesh(
        core_axis_name="c", subcore_axis_name="s", num_cores=_NC)

    @functools.partial(
        pl.kernel,
        mesh=mesh,
        out_type=[
            jax.ShapeDtypeStruct((n_out_pad,), jnp.float32),
            jax.ShapeDtypeStruct((_NC * n_out_pad,), jnp.float32),
            jax.ShapeDtypeStruct((_NC * n_out_pad,), jnp.float32),
        ],
        scratch_types=[
            pltpu.VMEM((ch, _CB), jnp.int32),
            pltpu.VMEM((ch, _CB), jnp.int32),
            pltpu.VMEM((ch, _CB), jnp.float32),
            pltpu.VMEM((_CB,), jnp.float32),
            pltpu.VMEM((zl,), jnp.float32),
            pltpu.VMEM((sl,), jnp.float32),
            pltpu.VMEM((sl,), jnp.float32),
            pltpu.VMEM((sl,), jnp.float32),
            pltpu.VMEM((sl,), jnp.float32),
            pltpu.VMEM((sl,), jnp.float32),
            pltpu.VMEM((_LANE,), jnp.float32),
            pltpu.VMEM_SHARED((n_out_pad,), jnp.float32),
            pltpu.VMEM_SHARED((n_out_pad,), jnp.float32),
            pltpu.VMEM_SHARED((n_out_pad,), jnp.float32),
            pltpu.SemaphoreType.DMA,
            pltpu.SemaphoreType.DMA,
            pltpu.SemaphoreType.REGULAR,
        ],
    )
    def sc_kernel(v_hbm, ei_hbm, bo_hbm,
                  out_hbm, sp_hbm, dp_hbm,
                  srcb, dstb, valb, ones, zbuf, s0b, s1b, d0b, d1b,
                  obuf, bob, v_sh, s_sh, d_sh, sem_g, sem_s, sem_c):
        c = lax.axis_index("c")
        s = lax.axis_index("s")
        t = c * _NS + s
        is0 = c == 0
        start = jnp.where(is0, s * ch0, _NS * ch0 + s * ch1)
        rounds = jnp.where(is0, ch0 // _K, ch1 // _K)

        for j in range(zl // _LANE):
            zbuf[pl.ds(j * _LANE, _LANE)] = jnp.zeros((_LANE,), jnp.float32)
        zbase = s * zl
        pltpu.sync_copy(zbuf, s_sh.at[pl.ds(zbase, zl)])
        pltpu.sync_copy(zbuf, d_sh.at[pl.ds(zbase, zl)])
        pltpu.sync_copy(v_hbm.at[pl.ds(zbase, zl)],
                        v_sh.at[pl.ds(zbase, zl)])

        for j in range(_CB // _LANE):
            ones[pl.ds(j * _LANE, _LANE)] = jnp.ones((_LANE,), jnp.float32)

        pltpu.sync_copy(ei_hbm.at[0, pl.ds(start, ch)], srcb)
        pltpu.sync_copy(ei_hbm.at[1, pl.ds(start, ch)], dstb)
        plsc.subcore_barrier()

        def fire_gather(q):
            pltpu.async_copy(v_sh.at[srcb.at[q]], valb.at[q], sem_g)

        def fire_deg_scatter(q):
            pltpu.async_copy(ones, d_sh.at[dstb.at[q]], sem_s, add=True)

        def fire_sum_scatter(q):
            pltpu.async_copy(valb.at[q], s_sh.at[dstb.at[q]], sem_s,
                             add=True)

        def drain(sem, count):
            for _ in range(count):
                pltpu.make_async_copy(
                    v_hbm.at[pl.ds(0, _CB)], valb.at[0], sem).wait()

        for j in range(_K):
            fire_gather(j)
            fire_deg_scatter(j)

        def round_body(r, carry):
            @pl.when(r + 1 < rounds)
            def _():
                for j in range(_K):
                    fire_gather((r + 1) * _K + j)
                    fire_deg_scatter((r + 1) * _K + j)
            drain(sem_g, _K)
            for j in range(_K):
                fire_sum_scatter(r * _K + j)
            @pl.when(r > 0)
            def _():
                drain(sem_s, 2 * _K)
            return carry

        lax.fori_loop(0, rounds, round_body, 0)
        drain(sem_s, 2 * _K)
        plsc.subcore_barrier()

        pltpu.sync_copy(s_sh.at[pl.ds(zbase, zl)],
                        sp_hbm.at[pl.ds(c * n_out_pad + zbase, zl)])
        pltpu.sync_copy(d_sh.at[pl.ds(zbase, zl)],
                        dp_hbm.at[pl.ds(c * n_out_pad + zbase, zl)])
        plsc.subcore_barrier()
        pltpu.core_barrier(sem_c, core_axis_name="c")

        base = t * sl
        pltpu.sync_copy(sp_hbm.at[pl.ds(base, sl)], s0b)
        pltpu.sync_copy(sp_hbm.at[pl.ds(n_out_pad + base, sl)], s1b)
        pltpu.sync_copy(dp_hbm.at[pl.ds(base, sl)], d0b)
        pltpu.sync_copy(dp_hbm.at[pl.ds(n_out_pad + base, sl)], d1b)
        pltpu.sync_copy(bo_hbm, bob)
        bias = bob[...]
        for j in range(sl // _LANE):
            ix = pl.ds(j * _LANE, _LANE)
            ssum = s0b[ix] + s1b[ix]
            dsum = d0b[ix] + d1b[ix]
            obuf[ix] = ssum / jnp.maximum(dsum, 1.0) + bias
        pltpu.sync_copy(obuf, out_hbm.at[pl.ds(base, sl)])

    return sc_kernel


def kernel(x, edge_index, W0, Wp, bp, Wu, bu, Wr, br, Wh, bh, Wo, bo):
    n, _ = x.shape
    e = edge_index.shape[1]

    blk = _NW * _LANE
    n_out_pad = ((n + blk - 1) // blk) * blk

    W1, v = _dense(x, W0, Wp, bp, Wu, bu, Wr, br, Wh, bh, Wo)
    v_flat = jnp.pad(v.reshape(n), (0, n_out_pad - n))

    def _ceil(a, b):
        return -(-a // b)

    nch = _ceil(e, _CB)
    ch0 = _ceil(_ceil(nch * 13, 20), _NS)
    ch0 = _ceil(ch0, _K) * _K
    ch1 = _ceil(max(nch - _NS * ch0, _NS), _NS)
    ch1 = _ceil(ch1, _K) * _K
    rows = _NS * (ch0 + ch1) + (ch0 - ch1)
    pad_e = rows * _CB - e
    ei = jnp.concatenate(
        [edge_index, jnp.full((2, pad_e), n, jnp.int32)], axis=1).reshape(
            2, rows, _CB)
    bo16 = jnp.broadcast_to(bo.astype(jnp.float32), (_LANE,))

    out_pad, _, _ = _make_sc(n_out_pad, ch0, ch1)(v_flat, ei, bo16)
    out = out_pad[:n].reshape(n, 1)
    return out, W1


# speedup vs baseline: 1.1907x; 1.1907x over previous
import functools

import jax
import jax.numpy as jnp
from jax import lax
from jax.experimental import pallas as pl
from jax.experimental.pallas import tpu as pltpu
from jax.experimental.pallas import tpu_sc as plsc


def _dense_body(x_ref, W0_ref, Wp_ref, bp_ref, Wu_ref, bu_ref, Wr_ref,
                br_ref, Wh_ref, bh_ref, Wo_ref, W1_ref, v_ref):
    x = x_ref[...]
    n = x.shape[0]
    din = x.shape[1]
    W0 = W0_ref[...]

    mean = jnp.sum(x, axis=0, keepdims=True) * (1.0 / n)
    ctx = mean @ Wp_ref[...] + bp_ref[...]

    Wu = Wu_ref[...]
    Wr = Wr_ref[...]
    Wh = Wh_ref[...]
    z = jax.nn.sigmoid(ctx @ Wu[:din] + W0 @ Wu[din:] + bu_ref[...])
    r = jax.nn.sigmoid(ctx @ Wr[:din] + W0 @ Wr[din:] + br_ref[...])
    h_tilde = jnp.tanh(ctx @ Wh[:din] + (r * W0) @ Wh[din:] + bh_ref[...])
    W1 = z * W0 + (1.0 - z) * h_tilde
    W1_ref[...] = W1

    Wv = W1 @ Wo_ref[...]
    v = x @ Wv
    pad = v_ref.shape[0] - n
    v_ref[...] = jnp.concatenate(
        [v, jnp.zeros((pad, 1), dtype=v.dtype)], axis=0)


def _dense(x, W0, Wp, bp, Wu, bu, Wr, br, Wh, bh, Wo, n_pad):
    din, dh = W0.shape
    return pl.pallas_call(
        _dense_body,
        out_shape=[
            jax.ShapeDtypeStruct((din, dh), jnp.float32),
            jax.ShapeDtypeStruct((n_pad, 1), jnp.float32),
        ],
    )(x, W0, Wp, bp.reshape(1, -1), Wu, bu.reshape(1, -1),
      Wr, br.reshape(1, -1), Wh, bh.reshape(1, -1), Wo)


_NC = 2
_NS = 16
_NW = _NC * _NS
_LANE = 16
_CB = 128
_K = 8


def _make_sc(n_out_pad, ch0, ch1):
    zl = n_out_pad // _NS
    sl = n_out_pad // _NW
    ch = ch0
    mesh = plsc.VectorSubcoreMesh(
        core_axis_name="c", subcore_axis_name="s", num_cores=_NC)

    @functools.partial(
        pl.kernel,
        mesh=mesh,
        out_type=[
            jax.ShapeDtypeStruct((n_out_pad,), jnp.float32),
            jax.ShapeDtypeStruct((_NC * n_out_pad,), jnp.float32),
            jax.ShapeDtypeStruct((_NC * n_out_pad,), jnp.float32),
        ],
        scratch_types=[
            pltpu.VMEM((ch, _CB), jnp.int32),
            pltpu.VMEM((ch, _CB), jnp.int32),
            pltpu.VMEM((ch, _CB), jnp.float32),
            pltpu.VMEM((_CB,), jnp.float32),
            pltpu.VMEM((zl,), jnp.float32),
            pltpu.VMEM((sl,), jnp.float32),
            pltpu.VMEM((sl,), jnp.float32),
            pltpu.VMEM((sl,), jnp.float32),
            pltpu.VMEM((sl,), jnp.float32),
            pltpu.VMEM((sl,), jnp.float32),
            pltpu.VMEM((_LANE,), jnp.float32),
            pltpu.VMEM_SHARED((n_out_pad,), jnp.float32),
            pltpu.VMEM_SHARED((n_out_pad,), jnp.float32),
            pltpu.VMEM_SHARED((n_out_pad,), jnp.float32),
            pltpu.SemaphoreType.DMA,
            pltpu.SemaphoreType.DMA,
            pltpu.SemaphoreType.REGULAR,
        ],
    )
    def sc_kernel(v_hbm, ei_hbm, bo_hbm,
                  out_hbm, sp_hbm, dp_hbm,
                  srcb, dstb, valb, ones, zbuf, s0b, s1b, d0b, d1b,
                  obuf, bob, v_sh, s_sh, d_sh, sem_g, sem_s, sem_c):
        c = lax.axis_index("c")
        s = lax.axis_index("s")
        t = c * _NS + s
        is0 = c == 0
        start = jnp.where(is0, s * ch0, _NS * ch0 + s * ch1)
        rounds = jnp.where(is0, ch0 // _K, ch1 // _K)

        for j in range(zl // _LANE):
            zbuf[pl.ds(j * _LANE, _LANE)] = jnp.zeros((_LANE,), jnp.float32)
        zbase = s * zl
        pltpu.sync_copy(zbuf, s_sh.at[pl.ds(zbase, zl)])
        pltpu.sync_copy(zbuf, d_sh.at[pl.ds(zbase, zl)])
        pltpu.sync_copy(v_hbm.at[pl.ds(zbase, zl)],
                        v_sh.at[pl.ds(zbase, zl)])

        for j in range(_CB // _LANE):
            ones[pl.ds(j * _LANE, _LANE)] = jnp.ones((_LANE,), jnp.float32)

        pltpu.sync_copy(ei_hbm.at[0, pl.ds(start, ch)], srcb)
        pltpu.sync_copy(ei_hbm.at[1, pl.ds(start, ch)], dstb)
        plsc.subcore_barrier()

        def fire_gather(q):
            pltpu.async_copy(v_sh.at[srcb.at[q]], valb.at[q], sem_g)

        def fire_deg_scatter(q):
            pltpu.async_copy(ones, d_sh.at[dstb.at[q]], sem_s, add=True)

        def fire_sum_scatter(q):
            pltpu.async_copy(valb.at[q], s_sh.at[dstb.at[q]], sem_s,
                             add=True)

        def drain(sem, count):
            for _ in range(count):
                pltpu.make_async_copy(
                    v_hbm.at[pl.ds(0, _CB)], valb.at[0], sem).wait()

        for j in range(_K):
            fire_gather(j)
            fire_deg_scatter(j)

        def round_body(r, carry):
            @pl.when(r + 1 < rounds)
            def _():
                for j in range(_K):
                    fire_gather((r + 1) * _K + j)
                    fire_deg_scatter((r + 1) * _K + j)
            drain(sem_g, _K)
            for j in range(_K):
                fire_sum_scatter(r * _K + j)
            @pl.when(r > 0)
            def _():
                drain(sem_s, 2 * _K)
            return carry

        lax.fori_loop(0, rounds, round_body, 0)
        drain(sem_s, 2 * _K)
        plsc.subcore_barrier()

        pltpu.sync_copy(s_sh.at[pl.ds(zbase, zl)],
                        sp_hbm.at[pl.ds(c * n_out_pad + zbase, zl)])
        pltpu.sync_copy(d_sh.at[pl.ds(zbase, zl)],
                        dp_hbm.at[pl.ds(c * n_out_pad + zbase, zl)])
        plsc.subcore_barrier()
        pltpu.core_barrier(sem_c, core_axis_name="c")

        base = t * sl
        pltpu.sync_copy(sp_hbm.at[pl.ds(base, sl)], s0b)
        pltpu.sync_copy(sp_hbm.at[pl.ds(n_out_pad + base, sl)], s1b)
        pltpu.sync_copy(dp_hbm.at[pl.ds(base, sl)], d0b)
        pltpu.sync_copy(dp_hbm.at[pl.ds(n_out_pad + base, sl)], d1b)
        pltpu.sync_copy(bo_hbm, bob)
        bias = bob[...]
        for j in range(sl // _LANE):
            ix = pl.ds(j * _LANE, _LANE)
            ssum = s0b[ix] + s1b[ix]
            dsum = d0b[ix] + d1b[ix]
            obuf[ix] = ssum / jnp.maximum(dsum, 1.0) + bias
        pltpu.sync_copy(obuf, out_hbm.at[pl.ds(base, sl)])

    return sc_kernel


def kernel(x, edge_index, W0, Wp, bp, Wu, bu, Wr, br, Wh, bh, Wo, bo):
    n, _ = x.shape
    e = edge_index.shape[1]

    blk = _NW * _LANE
    n_out_pad = ((n + blk - 1) // blk) * blk

    W1, v = _dense(x, W0, Wp, bp, Wu, bu, Wr, br, Wh, bh, Wo, n_out_pad)
    v_flat = v.reshape(n_out_pad)

    def _ceil(a, b):
        return -(-a // b)

    nch = _ceil(e, _CB)
    ch0 = _ceil(_ceil(nch * 13, 20), _NS)
    ch0 = _ceil(ch0, _K) * _K
    ch1 = _ceil(max(nch - _NS * ch0, _NS), _NS)
    ch1 = _ceil(ch1, _K) * _K
    rows = _NS * (ch0 + ch1) + (ch0 - ch1)
    pad_e = rows * _CB - e
    ei = jnp.concatenate(
        [edge_index, jnp.full((2, pad_e), n, jnp.int32)], axis=1).reshape(
            2, rows, _CB)
    bo16 = jnp.broadcast_to(bo.astype(jnp.float32), (_LANE,))

    out_pad, _, _ = _make_sc(n_out_pad, ch0, ch1)(v_flat, ei, bo16)
    out = out_pad[:n].reshape(n, 1)
    return out, W1


# speedup vs baseline: 1.2224x; 1.0267x over previous
import functools

import jax
import jax.numpy as jnp
from jax import lax
from jax.experimental import pallas as pl
from jax.experimental.pallas import tpu as pltpu
from jax.experimental.pallas import tpu_sc as plsc


def _dense_body(x_ref, W0_ref, Wp_ref, bp_ref, Wu_ref, bu_ref, Wr_ref,
                br_ref, Wh_ref, bh_ref, Wo_ref, W1_ref, v_ref):
    x = x_ref[...]
    n = x.shape[0]
    din = x.shape[1]
    W0 = W0_ref[...]

    mean = jnp.sum(x, axis=0, keepdims=True) * (1.0 / n)
    ctx = mean @ Wp_ref[...] + bp_ref[...]

    Wu = Wu_ref[...]
    Wr = Wr_ref[...]
    Wh = Wh_ref[...]
    z = jax.nn.sigmoid(ctx @ Wu[:din] + W0 @ Wu[din:] + bu_ref[...])
    r = jax.nn.sigmoid(ctx @ Wr[:din] + W0 @ Wr[din:] + br_ref[...])
    h_tilde = jnp.tanh(ctx @ Wh[:din] + (r * W0) @ Wh[din:] + bh_ref[...])
    W1 = z * W0 + (1.0 - z) * h_tilde
    W1_ref[...] = W1

    Wv = W1 @ Wo_ref[...]
    v = x @ Wv
    pad = v_ref.shape[0] - n
    v_ref[...] = jnp.concatenate(
        [v, jnp.zeros((pad, 1), dtype=v.dtype)], axis=0)


def _dense(x, W0, Wp, bp, Wu, bu, Wr, br, Wh, bh, Wo, n_pad):
    din, dh = W0.shape
    return pl.pallas_call(
        _dense_body,
        out_shape=[
            jax.ShapeDtypeStruct((din, dh), jnp.float32),
            jax.ShapeDtypeStruct((n_pad, 1), jnp.float32),
        ],
    )(x, W0, Wp, bp.reshape(1, -1), Wu, bu.reshape(1, -1),
      Wr, br.reshape(1, -1), Wh, bh.reshape(1, -1), Wo)


_NC = 2
_NS = 16
_NW = _NC * _NS
_LANE = 16
_CB = 128
_K = 16


def _make_sc(n_out_pad, ch0, ch1):
    zl = n_out_pad // _NS
    sl = n_out_pad // _NW
    ch = ch0
    mesh = plsc.VectorSubcoreMesh(
        core_axis_name="c", subcore_axis_name="s", num_cores=_NC)

    @functools.partial(
        pl.kernel,
        mesh=mesh,
        out_type=[
            jax.ShapeDtypeStruct((n_out_pad,), jnp.float32),
            jax.ShapeDtypeStruct((_NC * n_out_pad,), jnp.float32),
            jax.ShapeDtypeStruct((_NC * n_out_pad,), jnp.float32),
        ],
        scratch_types=[
            pltpu.VMEM((ch, _CB), jnp.int32),
            pltpu.VMEM((ch, _CB), jnp.int32),
            pltpu.VMEM((ch, _CB), jnp.float32),
            pltpu.VMEM((_CB,), jnp.float32),
            pltpu.VMEM((zl,), jnp.float32),
            pltpu.VMEM((sl,), jnp.float32),
            pltpu.VMEM((sl,), jnp.float32),
            pltpu.VMEM((sl,), jnp.float32),
            pltpu.VMEM((sl,), jnp.float32),
            pltpu.VMEM((sl,), jnp.float32),
            pltpu.VMEM((_LANE,), jnp.float32),
            pltpu.VMEM_SHARED((n_out_pad,), jnp.float32),
            pltpu.VMEM_SHARED((n_out_pad,), jnp.float32),
            pltpu.VMEM_SHARED((n_out_pad,), jnp.float32),
            pltpu.SemaphoreType.DMA,
            pltpu.SemaphoreType.DMA,
            pltpu.SemaphoreType.REGULAR,
        ],
    )
    def sc_kernel(v_hbm, ei_hbm, bo_hbm,
                  out_hbm, sp_hbm, dp_hbm,
                  srcb, dstb, valb, ones, zbuf, s0b, s1b, d0b, d1b,
                  obuf, bob, v_sh, s_sh, d_sh, sem_g, sem_s, sem_c):
        c = lax.axis_index("c")
        s = lax.axis_index("s")
        t = c * _NS + s
        is0 = c == 0
        start = jnp.where(is0, s * ch0, _NS * ch0 + s * ch1)
        rounds = jnp.where(is0, ch0 // _K, ch1 // _K)

        for j in range(zl // _LANE):
            zbuf[pl.ds(j * _LANE, _LANE)] = jnp.zeros((_LANE,), jnp.float32)
        zbase = s * zl
        pltpu.sync_copy(zbuf, s_sh.at[pl.ds(zbase, zl)])
        pltpu.sync_copy(zbuf, d_sh.at[pl.ds(zbase, zl)])
        pltpu.sync_copy(v_hbm.at[pl.ds(zbase, zl)],
                        v_sh.at[pl.ds(zbase, zl)])

        for j in range(_CB // _LANE):
            ones[pl.ds(j * _LANE, _LANE)] = jnp.ones((_LANE,), jnp.float32)

        pltpu.sync_copy(ei_hbm.at[0, pl.ds(start, ch)], srcb)
        pltpu.sync_copy(ei_hbm.at[1, pl.ds(start, ch)], dstb)
        plsc.subcore_barrier()

        def fire_gather(q):
            pltpu.async_copy(v_sh.at[srcb.at[q]], valb.at[q], sem_g)

        def fire_deg_scatter(q):
            pltpu.async_copy(ones, d_sh.at[dstb.at[q]], sem_s, add=True)

        def fire_sum_scatter(q):
            pltpu.async_copy(valb.at[q], s_sh.at[dstb.at[q]], sem_s,
                             add=True)

        def drain(sem, count):
            for _ in range(count):
                pltpu.make_async_copy(
                    v_hbm.at[pl.ds(0, _CB)], valb.at[0], sem).wait()

        for j in range(_K):
            fire_gather(j)
            fire_deg_scatter(j)

        def round_body(r, carry):
            @pl.when(r + 1 < rounds)
            def _():
                for j in range(_K):
                    fire_gather((r + 1) * _K + j)
                    fire_deg_scatter((r + 1) * _K + j)
            drain(sem_g, _K)
            for j in range(_K):
                fire_sum_scatter(r * _K + j)
            @pl.when(r > 0)
            def _():
                drain(sem_s, 2 * _K)
            return carry

        lax.fori_loop(0, rounds, round_body, 0)
        drain(sem_s, 2 * _K)
        plsc.subcore_barrier()

        pltpu.sync_copy(s_sh.at[pl.ds(zbase, zl)],
                        sp_hbm.at[pl.ds(c * n_out_pad + zbase, zl)])
        pltpu.sync_copy(d_sh.at[pl.ds(zbase, zl)],
                        dp_hbm.at[pl.ds(c * n_out_pad + zbase, zl)])
        plsc.subcore_barrier()
        pltpu.core_barrier(sem_c, core_axis_name="c")

        base = t * sl
        pltpu.sync_copy(sp_hbm.at[pl.ds(base, sl)], s0b)
        pltpu.sync_copy(sp_hbm.at[pl.ds(n_out_pad + base, sl)], s1b)
        pltpu.sync_copy(dp_hbm.at[pl.ds(base, sl)], d0b)
        pltpu.sync_copy(dp_hbm.at[pl.ds(n_out_pad + base, sl)], d1b)
        pltpu.sync_copy(bo_hbm, bob)
        bias = bob[...]
        for j in range(sl // _LANE):
            ix = pl.ds(j * _LANE, _LANE)
            ssum = s0b[ix] + s1b[ix]
            dsum = d0b[ix] + d1b[ix]
            obuf[ix] = ssum / jnp.maximum(dsum, 1.0) + bias
        pltpu.sync_copy(obuf, out_hbm.at[pl.ds(base, sl)])

    return sc_kernel


def kernel(x, edge_index, W0, Wp, bp, Wu, bu, Wr, br, Wh, bh, Wo, bo):
    n, _ = x.shape
    e = edge_index.shape[1]

    blk = _NW * _LANE
    n_out_pad = ((n + blk - 1) // blk) * blk

    W1, v = _dense(x, W0, Wp, bp, Wu, bu, Wr, br, Wh, bh, Wo, n_out_pad)
    v_flat = v.reshape(n_out_pad)

    def _ceil(a, b):
        return -(-a // b)

    nch = _ceil(e, _CB)
    ch0 = _ceil(_ceil(nch * 13, 20), _NS)
    ch0 = _ceil(ch0, _K) * _K
    ch1 = _ceil(max(nch - _NS * ch0, _NS), _NS)
    ch1 = _ceil(ch1, _K) * _K
    rows = _NS * (ch0 + ch1) + (ch0 - ch1)
    pad_e = rows * _CB - e
    ei = jnp.concatenate(
        [edge_index, jnp.full((2, pad_e), n, jnp.int32)], axis=1).reshape(
            2, rows, _CB)
    bo16 = jnp.broadcast_to(bo.astype(jnp.float32), (_LANE,))

    out_pad, _, _ = _make_sc(n_out_pad, ch0, ch1)(v_flat, ei, bo16)
    out = out_pad[:n].reshape(n, 1)
    return out, W1
